# baseline (speedup 1.0000x reference)
import jax
import jax.numpy as jnp
from jax import lax
from jax.experimental import pallas as pl
from jax.experimental.pallas import tpu as pltpu
from jax.experimental.pallas import tpu_sc as plsc

DIM = 1024
E = 8
EPS = 1e-09
THRESHOLD = 0.2
CAP = 320
TB = 128
B = 4
N = 2048
NC = N // TB
NCHUNK = B * NC
ROW = E * CAP


def _route_kernel(x_ref, w_ref, idx_ref, val_ref, rec_ref, carry_ref,
                  tri_ref):
    bb = pl.program_id(0)
    j = pl.program_id(1)

    @pl.when(jnp.logical_and(bb == 0, j == 0))
    def _init_tri():
        r = lax.broadcasted_iota(jnp.int32, (TB, TB), 0)
        c = lax.broadcasted_iota(jnp.int32, (TB, TB), 1)
        tri_ref[...] = (r < c).astype(jnp.float32)

    @pl.when(bb > 0)
    def _phase2():
        rec = rec_ref[:, pl.ds(j * TB, TB)]
        i1f = rec[0:1, :]
        p1eff = rec[1:2, :]
        v1 = rec[2:3, :]
        i2f = rec[3:4, :]
        g2n = rec[4:5, :]
        prefix2 = rec[5:6, :]
        flag2 = rec[6:7, :]

        count1 = carry_ref[0:E, 1:2]
        iotaE = lax.broadcasted_iota(jnp.int32, (E, TB), 0)
        c_i2 = jnp.sum(
            jnp.where(iotaE == i2f.astype(jnp.int32), count1, 0.0),
            axis=0, keepdims=True)
        pos2 = (prefix2 + c_i2) * flag2
        ok2 = (pos2 < float(CAP)).astype(jnp.float32)
        v2 = g2n * flag2 * ok2
        p2eff = pos2 * ok2

        tok = lax.broadcasted_iota(jnp.int32, (1, TB), 1)
        base = ((bb - 1) * N + j * TB + tok) * ROW
        k1g = base + i1f.astype(jnp.int32) * CAP + p1eff.astype(jnp.int32)
        k2g = base + i2f.astype(jnp.int32) * CAP + p2eff.astype(jnp.int32)

        idx_ref[0] = jnp.concatenate([k1g, k2g], axis=0)
        val_ref[0] = jnp.concatenate([v1, v2], axis=0)

    @pl.when(bb < B)
    def _phase1():
        xb = x_ref[0]
        w = w_ref[...]
        logits = jnp.dot(xb, w, preferred_element_type=jnp.float32)
        lt = logits.T

        iotaE = lax.broadcasted_iota(jnp.int32, (E, TB), 0)
        m = jnp.max(lt, axis=0, keepdims=True)
        ex = jnp.exp(lt - m)
        g = ex / jnp.sum(ex, axis=0, keepdims=True)

        g1v = jnp.max(g, axis=0, keepdims=True)
        i1 = jnp.min(jnp.where(g == g1v, iotaE, E), axis=0, keepdims=True)
        mask1 = (iotaE == i1).astype(jnp.float32)

        g_wo = g * (1.0 - mask1)
        g2v = jnp.max(g_wo, axis=0, keepdims=True)
        i2 = jnp.min(jnp.where(g_wo == g2v, iotaE, E), axis=0, keepdims=True)

        g1n = g1v / (g1v + g2v + EPS)
        g2n = g2v / (g1n + g2v + EPS)

        mask2 = (iotaE == i2).astype(jnp.float32) * (
            g2n > THRESHOLD).astype(jnp.float32)

        M = jnp.concatenate([mask1, mask2], axis=0)
        excl = jnp.dot(M, tri_ref[...],
                       preferred_element_type=jnp.float32)

        base = jnp.where(j == 0, 0.0, carry_ref[:, 0:1])
        totals = base + jnp.sum(M, axis=1, keepdims=True)
        carry_ref[:, 0:1] = totals

        base1, base2 = base[:E, :], base[E:, :]
        excl1, excl2 = excl[:E, :], excl[E:, :]

        pos1 = (base1 + excl1) * mask1
        mask1k = mask1 * (pos1 < float(CAP)).astype(jnp.float32)
        flat1 = jnp.sum(mask1k, axis=0, keepdims=True)
        p1eff = jnp.sum(pos1 * mask1k, axis=0, keepdims=True)
        v1 = g1n * flat1

        prefix2 = jnp.sum((base2 + excl2) * mask2, axis=0, keepdims=True)
        flag2 = jnp.sum(mask2, axis=0, keepdims=True)

        @pl.when(j == NC - 1)
        def _store_count1():
            carry_ref[0:E, 1:2] = jnp.minimum(totals[:E, :], float(CAP))

        rec_ref[:, pl.ds(j * TB, TB)] = jnp.concatenate(
            [i1.astype(jnp.float32), p1eff, v1, i2.astype(jnp.float32),
             g2n, prefix2, flag2, jnp.zeros((1, TB), jnp.float32)], axis=0)


def _routing_records(x, gating_weights):
    b, n, d = x.shape
    return pl.pallas_call(
        _route_kernel,
        grid=(b + 1, NC),
        in_specs=[
            pl.BlockSpec((1, TB, d),
                         lambda bb, j: (jnp.minimum(bb, B - 1),
                                        jnp.where(bb < B, j, 0), 0)),
            pl.BlockSpec((d, E), lambda bb, j: (0, 0)),
        ],
        out_specs=[
            pl.BlockSpec((1, 2, TB),
                         lambda bb, j: (jnp.where(bb > 0,
                                                  (bb - 1) * NC + j, 0),
                                        0, 0)),
            pl.BlockSpec((1, 2, TB),
                         lambda bb, j: (jnp.where(bb > 0,
                                                  (bb - 1) * NC + j, 0),
                                        0, 0)),
        ],
        out_shape=[
            jax.ShapeDtypeStruct((NCHUNK, 2, TB), jnp.int32),
            jax.ShapeDtypeStruct((NCHUNK, 2, TB), jnp.float32),
        ],
        scratch_shapes=[
            pltpu.VMEM((E, N), jnp.float32),
            pltpu.VMEM((2 * E, 128), jnp.float32),
            pltpu.VMEM((TB, TB), jnp.float32),
        ],
    )(x, gating_weights)


_NUM_SC = 2
_NUM_SUB = 16
_NW = _NUM_SC * _NUM_SUB
_CPW = NCHUNK // _NW


def _scatter_body(buf_ref, idx_hbm, val_hbm, idx_v, val_v, sem):
    wid = lax.axis_index("s") * _NUM_SC + lax.axis_index("c")
    for cc in range(_CPW):
        chunk = wid * _CPW + cc
        pltpu.sync_copy(idx_hbm.at[chunk], idx_v.at[cc])
        pltpu.sync_copy(val_hbm.at[chunk], val_v.at[cc])
    for cc in range(_CPW):
        for q in range(2):
            pltpu.async_copy(val_v.at[cc, q],
                             buf_ref.at[idx_v.at[cc, q]], sem).wait()


def _make_scatter_kernel():
    return pl.kernel(
        _scatter_body,
        out_type=(),
        mesh=plsc.VectorSubcoreMesh(core_axis_name="c",
                                    subcore_axis_name="s",
                                    num_cores=_NUM_SC,
                                    num_subcores=_NUM_SUB),
        scratch_types=[
            pltpu.VMEM((_CPW, 2, TB), jnp.int32),
            pltpu.VMEM((_CPW, 2, TB), jnp.float32),
            pltpu.SemaphoreType.DMA,
        ],
    )


def kernel(x, gating_weights):
    b, n, d = x.shape
    idx, val = _routing_records(x, gating_weights)
    buf = jax.new_ref(jnp.zeros((b * n * ROW,), jnp.float32))
    _make_scatter_kernel()(buf, idx, val)
    return buf[...].reshape(b, n, E, CAP)
